# baseline (speedup 1.0000x reference)
import functools

import jax
import jax.numpy as jnp
from jax import lax
from jax.experimental import pallas as pl
from jax.experimental.pallas import tpu as pltpu
from jax.experimental.pallas import tpu_sc as plsc

VOCAB1 = 1000001
EMBED_DIM = 64
BATCH = 4096
SEQ = 200

NC = 2
NS = 16
NW = NC * NS

LANES = 128
N_ITEMS = SEQ
R = 4
LA = 2
N_BLOCKS = N_ITEMS // R


def _build_kernel():
    mesh = plsc.VectorSubcoreMesh(core_axis_name="c", subcore_axis_name="s")

    @functools.partial(
        pl.kernel,
        mesh=mesh,
        out_type=jax.ShapeDtypeStruct((SEQ, 8, NW, 8, LANES), jnp.float32),
        compiler_params=pltpu.CompilerParams(
            use_tc_tiling_on_sc=False, needs_layout_passes=False
        ),
        scratch_types=[
            pltpu.VMEM((N_ITEMS, LANES), jnp.int32),
            pltpu.VMEM((R, LANES, EMBED_DIM), jnp.float32),
            pltpu.VMEM((R, 8, 8, LANES), jnp.float32),
        ]
        + [pltpu.SemaphoreType.DMA] * (2 * R),
    )
    def k(table_hbm, ids_t_hbm, out_hbm, idx_v, rows_v, trows_v, *sems):
        gsems = sems[:R]
        ssems = sems[R:]
        wid = lax.axis_index("s") * NC + lax.axis_index("c")
        pltpu.sync_copy(ids_t_hbm.at[:, pl.ds(wid * LANES, LANES)], idx_v)

        iota = lax.iota(jnp.int32, 16)
        row_idx = [iota + 16 * j for j in range(LANES // 16)]

        def fire_gather(g, s):
            pltpu.async_copy(table_hbm.at[idx_v.at[g]], rows_v.at[s], gsems[s])

        def wait_gather(g, s):
            pltpu.make_async_copy(
                table_hbm.at[idx_v.at[g]], rows_v.at[s], gsems[s]
            ).wait()

        def fire_store(g, s):
            pltpu.async_copy(trows_v.at[s], out_hbm.at[g, :, wid], ssems[s])

        def drain_store(s):
            pltpu.make_async_copy(
                trows_v.at[s], out_hbm.at[0, :, wid], ssems[s]
            ).wait()

        def transpose_item(s):
            rows = rows_v.at[s]

            def tr_body(q2, carry):
                for dq in range(2):
                    q = 2 * q2 + dq
                    col = jnp.full((16,), q, jnp.int32)
                    for j in range(LANES // 16):
                        val = plsc.load_gather(rows, [row_idx[j], col])
                        trows_v[s, q >> 3, q & 7, pl.ds(16 * j, 16)] = val
                return carry

            lax.fori_loop(0, 32, tr_body, 0)

        for g0 in range(LA):
            fire_gather(g0, g0)
        for p in range(R):
            fire_gather(p + LA, (p + LA) % R)
            wait_gather(p, p)
            transpose_item(p)
            fire_store(p, p)

        def body(i, carry):
            for p in range(R):
                g = i * R + p
                fire_gather(g + LA, (p + LA) % R)
                wait_gather(g, p)
                drain_store(p)
                transpose_item(p)
                fire_store(g, p)
            return carry

        lax.fori_loop(1, N_BLOCKS - 1, body, 0)

        base = (N_BLOCKS - 1) * R
        for p in range(R):
            g = base + p
            if p < R - LA:
                fire_gather(g + LA, (p + LA) % R)
            wait_gather(g, p)
            drain_store(p)
            transpose_item(p)
            fire_store(g, p)
        for p in range(R):
            drain_store(p)

    return k


_k = _build_kernel()


@jax.jit
def kernel(input_ids, table):
    ids_t = jnp.swapaxes(input_ids, 0, 1)
    out5 = _k(table, ids_t)
    return out5.transpose(2, 4, 0, 1, 3).reshape(BATCH, SEQ, EMBED_DIM)


# speedup vs baseline: 1.4562x; 1.4562x over previous
import functools

import jax
import jax.numpy as jnp
from jax import lax
from jax.experimental import pallas as pl
from jax.experimental.pallas import tpu as pltpu
from jax.experimental.pallas import tpu_sc as plsc

VOCAB1 = 1000001
EMBED_DIM = 64
BATCH = 4096
SEQ = 200

NC = 2
NS = 16
NW = NC * NS

LANES = 128
N_ITEMS = SEQ
R = 4
LA = 2
N_BLOCKS = N_ITEMS // R


def _build_kernel():
    mesh = plsc.VectorSubcoreMesh(core_axis_name="c", subcore_axis_name="s")

    @functools.partial(
        pl.kernel,
        mesh=mesh,
        out_type=jax.ShapeDtypeStruct((SEQ, 8, NW, 8, LANES), jnp.float32),
        compiler_params=pltpu.CompilerParams(
            use_tc_tiling_on_sc=False, needs_layout_passes=False
        ),
        scratch_types=[
            pltpu.VMEM((N_ITEMS, LANES), jnp.int32),
            pltpu.VMEM((R, LANES, EMBED_DIM), jnp.float32),
            pltpu.VMEM((R, 8, 8, LANES), jnp.float32),
        ]
        + [pltpu.SemaphoreType.DMA] * (2 * R),
    )
    def k(table_hbm, ids_t_hbm, out_hbm, idx_v, rows_v, trows_v, *sems):
        gsems = sems[:R]
        ssems = sems[R:]
        wid = lax.axis_index("s") * NC + lax.axis_index("c")
        pltpu.sync_copy(ids_t_hbm.at[:, pl.ds(wid * LANES, LANES)], idx_v)

        iota = lax.iota(jnp.int32, 16)
        row_idx = [iota + 16 * j for j in range(LANES // 16)]

        def fire_gather(g, s):
            pltpu.async_copy(table_hbm.at[idx_v.at[g]], rows_v.at[s], gsems[s])

        def wait_gather(g, s):
            pltpu.make_async_copy(
                table_hbm.at[idx_v.at[g]], rows_v.at[s], gsems[s]
            ).wait()

        def fire_store(g, s):
            pltpu.async_copy(trows_v.at[s], out_hbm.at[g, :, wid], ssems[s])

        def drain_store(s):
            pltpu.make_async_copy(
                trows_v.at[s], out_hbm.at[0, :, wid], ssems[s]
            ).wait()

        def transpose_item(s):
            rows = rows_v.at[s]

            @plsc.parallel_loop(0, EMBED_DIM, unroll=4)
            def _(q):
                col = jnp.full((16,), q, jnp.int32)
                for j in range(LANES // 16):
                    val = plsc.load_gather(rows, [row_idx[j], col])
                    trows_v[s, q >> 3, q & 7, pl.ds(16 * j, 16)] = val

        for g0 in range(LA):
            fire_gather(g0, g0)
        for p in range(R):
            fire_gather(p + LA, (p + LA) % R)
            wait_gather(p, p)
            transpose_item(p)
            fire_store(p, p)

        def body(i, carry):
            for p in range(R):
                g = i * R + p
                fire_gather(g + LA, (p + LA) % R)
                wait_gather(g, p)
                drain_store(p)
                transpose_item(p)
                fire_store(g, p)
            return carry

        lax.fori_loop(1, N_BLOCKS - 1, body, 0)

        base = (N_BLOCKS - 1) * R
        for p in range(R):
            g = base + p
            if p < R - LA:
                fire_gather(g + LA, (p + LA) % R)
            wait_gather(g, p)
            drain_store(p)
            transpose_item(p)
            fire_store(g, p)
        for p in range(R):
            drain_store(p)

    return k


_k = _build_kernel()


@jax.jit
def kernel(input_ids, table):
    ids_t = jnp.swapaxes(input_ids, 0, 1)
    out5 = _k(table, ids_t)
    return out5.transpose(2, 4, 0, 1, 3).reshape(BATCH, SEQ, EMBED_DIM)
